# baseline (speedup 1.0000x reference)
import functools

import jax
import jax.numpy as jnp
from jax import lax
from jax.experimental import pallas as pl
from jax.experimental.pallas import tpu as pltpu
from jax.experimental.pallas import tpu_sc as plsc

_NC = 2
_NS = 16
_NW = _NC * _NS

_CHUNK = 128
_NBUF = 2


def _conv_body(embt_hbm, tail_hbm, out_hbm, tin0, tin1, tout0, tout1,
               tail_in, rsems, wsems, *, n_blk, blk_per_w, n_tail):
  tins = (tin0, tin1)
  touts = (tout0, tout1)
  wid = lax.axis_index("s") * _NC + lax.axis_index("c")
  c0 = wid * blk_per_w
  lane = lax.iota(jnp.int32, 16)
  dcol = [lane + 16 * d16 for d16 in range(4)]

  def fire_read(c, b):
    pltpu.async_copy(embt_hbm.at[:, pl.ds(c * 128, 128)], tins[b],
                     rsems.at[b])

  def transpose_block(src, dst, nv):
    @plsc.parallel_loop(0, nv, unroll=4)
    def vrow(v):
      v_vec = jnp.full((16,), v, jnp.int32)
      base = pl.multiple_of(v * 64, 8)
      for d16 in range(4):
        vec = plsc.load_gather(src, [dcol[d16], v_vec])
        dst[pl.ds(base + 16 * d16, 16)] = vec

  def do_block(c, b):
    pltpu.make_async_copy(embt_hbm.at[:, pl.ds(c * 128, 128)], tins[b],
                          rsems.at[b]).wait()
    transpose_block(tins[b], touts[b], 128)
    pltpu.async_copy(touts[b], out_hbm.at[pl.ds(c * 8192, 8192)],
                     wsems.at[b])

  def wait_write(c, b):
    pltpu.make_async_copy(touts[b], out_hbm.at[pl.ds(c * 8192, 8192)],
                          wsems.at[b]).wait()

  for b in range(_NBUF):
    fire_read(c0 + b, b)
  for b in range(_NBUF):
    do_block(c0 + b, b)
    fire_read(c0 + b + _NBUF, b)

  def loop_body(i, _):
    j0 = i * _NBUF
    for b in range(_NBUF):
      j = j0 + b
      wait_write(c0 + j - _NBUF, b)
      do_block(c0 + j, b)
      fire_read(c0 + j + _NBUF, b)
    return ()

  lax.fori_loop(1, (blk_per_w - _NBUF) // _NBUF, loop_body, ())

  for t in range(_NBUF):
    j = blk_per_w - _NBUF + t
    wait_write(c0 + j - _NBUF, t)
    do_block(c0 + j, t)
  for t in range(_NBUF):
    wait_write(c0 + blk_per_w - _NBUF + t, t)

  n_rem = n_blk - blk_per_w * _NW

  @pl.when(wid == _NW - 1)
  def _():
    for r in range(n_rem):
      c = n_blk - n_rem + r
      b = r % _NBUF
      fire_read(c, b)
      do_block(c, b)
      wait_write(c, b)
    if n_tail:
      pltpu.sync_copy(tail_hbm, tail_in)
      transpose_block(tail_in, touts[0], n_tail)
      pltpu.sync_copy(touts[0].at[pl.ds(0, n_tail * 64)],
                      out_hbm.at[pl.ds(n_blk * 8192, n_tail * 64)])


def _gather_body(idx_hbm, table_hbm, out_hbm, idx_v, rows_v, tiles_v, gsems,
                 wsems, *, seq, bt_per_w):
  wid = lax.axis_index("s") * _NC + lax.axis_index("c")
  n_units = seq * bt_per_w

  pltpu.sync_copy(idx_hbm.at[:, pl.ds(wid * bt_per_w * _CHUNK,
                                      bt_per_w * _CHUNK)], idx_v)

  lane = lax.iota(jnp.int32, 16)
  row_idx = [lane + 16 * k for k in range(8)]

  def unit_su(u):
    return u // bt_per_w, u % bt_per_w

  def fire_gather(u, b):
    s, k = unit_su(u)
    pltpu.async_copy(
        table_hbm.at[idx_v.at[s, pl.ds(k * _CHUNK, _CHUNK)]],
        rows_v.at[b], gsems.at[b])

  def transpose_and_write(u, b):
    s, k = unit_su(u)
    pltpu.make_async_copy(
        table_hbm.at[idx_v.at[s, pl.ds(k * _CHUNK, _CHUNK)]],
        rows_v.at[b], gsems.at[b]).wait()

    @plsc.parallel_loop(0, 64, unroll=4)
    def col(d):
      d_vec = jnp.full((16,), d, jnp.int32)
      base = pl.multiple_of(d * _CHUNK, 8)
      for k2 in range(8):
        v = plsc.load_gather(rows_v.at[b], [row_idx[k2], d_vec])
        tiles_v[b, pl.ds(base + 16 * k2, 16)] = v

    for d_t in range(8):
      pltpu.async_copy(tiles_v.at[b, pl.ds(d_t * 1024, 1024)],
                       out_hbm.at[s, d_t, wid * bt_per_w + k], wsems.at[b])

  def wait_write(u, b):
    s, k = unit_su(u)
    for d_t in range(8):
      pltpu.make_async_copy(tiles_v.at[b, pl.ds(d_t * 1024, 1024)],
                            out_hbm.at[s, d_t, wid * bt_per_w + k],
                            wsems.at[b]).wait()

  for b in range(_NBUF):
    fire_gather(b, b)
  for b in range(_NBUF):
    transpose_and_write(b, b)
    fire_gather(b + _NBUF, b)

  def loop_body(i, _):
    u0 = i * _NBUF
    for b in range(_NBUF):
      u = u0 + b
      wait_write(u - _NBUF, b)
      transpose_and_write(u, b)
      fire_gather(u + _NBUF, b)
    return ()

  lax.fori_loop(1, (n_units - _NBUF) // _NBUF, loop_body, ())

  for t in range(_NBUF):
    u = n_units - _NBUF + t
    wait_write(u - _NBUF, t)
    transpose_and_write(u, t)
  for t in range(_NBUF):
    wait_write(n_units - _NBUF + t, t)


def kernel(X, emb):
  B, S = X.shape
  V, D = emb.shape
  assert D == 64 and B % (_NW * _CHUNK) == 0
  n_bt = B // _CHUNK
  bt_per_w = n_bt // _NW
  n_blk = V // 128
  blk_per_w = (n_blk // _NW) // _NBUF * _NBUF
  n_tail = V - n_blk * 128

  idx = X.T.astype(jnp.int32)
  embt = emb.T
  t0 = n_blk * 128 if n_tail else 0
  tail = jnp.pad(lax.slice(emb, (t0, 0), (t0 + max(n_tail, 1), D)).T,
                 ((0, 0), (0, 128 - max(n_tail, 1))))

  mesh = plsc.VectorSubcoreMesh(core_axis_name="c", subcore_axis_name="s")

  conv = functools.partial(_conv_body, n_blk=n_blk, blk_per_w=blk_per_w,
                           n_tail=n_tail)
  table_flat = pl.kernel(
      conv,
      out_type=jax.ShapeDtypeStruct((V * D,), jnp.float32),
      mesh=mesh,
      compiler_params=pltpu.CompilerParams(use_tc_tiling_on_sc=True,
                                           needs_layout_passes=False),
      scratch_types=[
          pltpu.VMEM((D, 128), jnp.float32),
          pltpu.VMEM((D, 128), jnp.float32),
          pltpu.VMEM((128 * D,), jnp.float32),
          pltpu.VMEM((128 * D,), jnp.float32),
          pltpu.VMEM((D, 128), jnp.float32),
          pltpu.SemaphoreType.DMA((_NBUF,)),
          pltpu.SemaphoreType.DMA((_NBUF,)),
      ],
  )(embt, tail)
  table = table_flat.reshape(V, D)

  body = functools.partial(_gather_body, seq=S, bt_per_w=bt_per_w)
  out5 = pl.kernel(
      body,
      out_type=jax.ShapeDtypeStruct((S, 8, n_bt, 8 * _CHUNK), jnp.float32),
      mesh=mesh,
      compiler_params=pltpu.CompilerParams(use_tc_tiling_on_sc=False,
                                           needs_layout_passes=False),
      scratch_types=[
          pltpu.VMEM((S, bt_per_w * _CHUNK), jnp.int32),
          pltpu.VMEM((_NBUF, _CHUNK, D), jnp.float32),
          pltpu.VMEM((_NBUF, 64 * _CHUNK), jnp.float32),
          pltpu.SemaphoreType.DMA((_NBUF,)),
          pltpu.SemaphoreType.DMA((_NBUF,)),
      ],
  )(idx, table)
  out5 = out5.reshape(S, 8, n_bt, 8, _CHUNK)
  return out5.transpose((2, 4, 0, 1, 3)).reshape(B, S, D)


# speedup vs baseline: 1.2327x; 1.2327x over previous
import functools

import jax
import jax.numpy as jnp
from jax import lax
from jax.experimental import pallas as pl
from jax.experimental.pallas import tpu as pltpu
from jax.experimental.pallas import tpu_sc as plsc

_NC = 2
_NS = 16
_NW = _NC * _NS

_CHUNK = 128
_NBUF = 2


def _body(idx_hbm, table_hbm, out_hbm, idx_v, rows_v, tiles_v, gsems, wsems,
          *, seq, bt_per_w):
  wid = lax.axis_index("s") * _NC + lax.axis_index("c")
  n_units = seq * bt_per_w

  pltpu.sync_copy(idx_hbm.at[:, pl.ds(wid * bt_per_w * _CHUNK,
                                      bt_per_w * _CHUNK)], idx_v)

  lane = lax.iota(jnp.int32, 16)
  row_idx = [lane + 16 * k for k in range(8)]

  def unit_su(u):
    return u // bt_per_w, u % bt_per_w

  def fire_gather(u, b):
    s, k = unit_su(u)
    pltpu.async_copy(
        table_hbm.at[idx_v.at[s, pl.ds(k * _CHUNK, _CHUNK)]],
        rows_v.at[b], gsems.at[b])

  def transpose_and_write(u, b):
    s, k = unit_su(u)
    pltpu.make_async_copy(
        table_hbm.at[idx_v.at[s, pl.ds(k * _CHUNK, _CHUNK)]],
        rows_v.at[b], gsems.at[b]).wait()

    @plsc.parallel_loop(0, 64, unroll=4)
    def col(d):
      d_vec = jnp.full((16,), d, jnp.int32)
      base = pl.multiple_of(d * _CHUNK, 8)
      for k in range(8):
        v = plsc.load_gather(rows_v.at[b], [row_idx[k], d_vec])
        tiles_v[b, pl.ds(base + 16 * k, 16)] = v
    for d_t in range(8):
      pltpu.async_copy(tiles_v.at[b, pl.ds(d_t * 1024, 1024)],
                       out_hbm.at[s, d_t, wid * bt_per_w + k], wsems.at[b])

  def wait_write(u, b):
    s, k = unit_su(u)
    for d_t in range(8):
      pltpu.make_async_copy(tiles_v.at[b, pl.ds(d_t * 1024, 1024)],
                            out_hbm.at[s, d_t, wid * bt_per_w + k],
                            wsems.at[b]).wait()

  for b in range(_NBUF):
    fire_gather(b, b)
  for b in range(_NBUF):
    transpose_and_write(b, b)
    fire_gather(b + _NBUF, b)

  def loop_body(i, _):
    u0 = i * _NBUF
    for b in range(_NBUF):
      u = u0 + b
      wait_write(u - _NBUF, b)
      transpose_and_write(u, b)
      fire_gather(u + _NBUF, b)
    return ()

  lax.fori_loop(1, (n_units - _NBUF) // _NBUF, loop_body, ())

  for t in range(_NBUF):
    u = n_units - _NBUF + t
    wait_write(u - _NBUF, t)
    transpose_and_write(u, t)
  for t in range(_NBUF):
    wait_write(n_units - _NBUF + t, t)


def kernel(X, emb):
  B, S = X.shape
  V, D = emb.shape
  assert D == 64 and B % (_NW * _CHUNK) == 0
  n_bt = B // _CHUNK
  bt_per_w = n_bt // _NW

  idx = X.T.astype(jnp.int32)
  emb128 = jnp.pad(emb, ((0, 0), (0, 128 - D)))

  mesh = plsc.VectorSubcoreMesh(core_axis_name="c", subcore_axis_name="s")
  body = functools.partial(_body, seq=S, bt_per_w=bt_per_w)
  out5 = pl.kernel(
      body,
      out_type=jax.ShapeDtypeStruct((S, 8, n_bt, 8 * _CHUNK), jnp.float32),
      mesh=mesh,
      compiler_params=pltpu.CompilerParams(use_tc_tiling_on_sc=False,
                                           needs_layout_passes=False),
      scratch_types=[
          pltpu.VMEM((S, bt_per_w * _CHUNK), jnp.int32),
          pltpu.VMEM((_NBUF, _CHUNK, 128), jnp.float32),
          pltpu.VMEM((_NBUF, 8 * 8 * _CHUNK), jnp.float32),
          pltpu.SemaphoreType.DMA((_NBUF,)),
          pltpu.SemaphoreType.DMA((_NBUF,)),
      ],
  )(idx, emb128)
  out5 = out5.reshape(S, 8, n_bt, 8, _CHUNK)
  return out5.transpose((2, 4, 0, 1, 3)).reshape(B, S, D)
